# baseline (speedup 1.0000x reference)
import functools

import jax
import jax.numpy as jnp
from jax.experimental import pallas as pl
from jax.experimental.pallas import tpu as pltpu


def _body(x_ref, u_ref, wm_ref, o_ref, *, TB, N, U):
    x = x_ref[0, 0, :]
    s_iota = jax.lax.broadcasted_iota(jnp.int32, (N, TB), 0)
    eq = s_iota == x[None, :]
    onehot_bf = jnp.where(eq, jnp.float32(1.0), jnp.float32(0.0)).astype(jnp.bfloat16)
    ut_bf = jnp.transpose(u_ref[...], (1, 0)).astype(jnp.bfloat16)
    a = jnp.broadcast_to(onehot_bf[:, None, :], (N, U, TB)).reshape(N * U, TB)
    b = jnp.broadcast_to(ut_bf[None, :, :], (N, U, TB)).reshape(N * U, TB)
    zt = jnp.concatenate([a * b, onehot_bf], axis=0)
    dn = (((0,), (0,)), ((), ()))
    logits = jax.lax.dot_general(zt, wm_ref[...], dn,
                                 preferred_element_type=jnp.float32)
    m = jnp.max(logits, axis=1, keepdims=True)
    ex = jnp.exp(logits - m)
    lz = jnp.log(jnp.sum(ex, axis=1, keepdims=True)) + m
    o_ref[...] = logits - lz


@functools.partial(jax.jit, static_argnames=("interpret", "tb"))
def kernel(x_curr, u_curr, logP0, W, interpret=False, tb=2048):
    T = x_curr.shape[0]
    N = logP0.shape[0]
    U = u_curr.shape[1]
    cols = jnp.arange(N)[None, :]
    srows = jnp.arange(N)[:, None]
    k = jnp.clip(cols - (cols > srows).astype(jnp.int32), 0, N - 2)
    W64 = jnp.take_along_axis(W, k[:, :, None], axis=1)
    W64 = jnp.where((cols == srows)[:, :, None], 0.0, W64)
    Wm = W64.transpose(0, 2, 1).reshape(N * U, N)
    Wtot = jnp.concatenate([Wm, logP0], axis=0).astype(jnp.bfloat16)

    TB = tb
    NB = T // TB
    x3 = x_curr.astype(jnp.int32).reshape(NB, 1, TB)
    out = pl.pallas_call(
        functools.partial(_body, TB=TB, N=N, U=U),
        grid=(NB,),
        in_specs=[
            pl.BlockSpec((1, 1, TB), lambda i: (i, 0, 0)),
            pl.BlockSpec((TB, U), lambda i: (i, 0)),
            pl.BlockSpec((N * U + N, N), lambda i: (0, 0)),
        ],
        out_specs=pl.BlockSpec((TB, N), lambda i: (i, 0)),
        out_shape=jax.ShapeDtypeStruct((T, N), jnp.float32),
        compiler_params=pltpu.CompilerParams(
            dimension_semantics=("arbitrary",),
        ),
        interpret=interpret,
    )(x3, u_curr, Wtot)
    return out


# speedup vs baseline: 1.2069x; 1.2069x over previous
import functools

import jax
import jax.numpy as jnp
from jax.experimental import pallas as pl
from jax.experimental.pallas import tpu as pltpu


def _body(x_ref, ut_ref, wm_ref, o_ref, *, TB, N, U):
    x = x_ref[0, 0, :]
    s_iota = jax.lax.broadcasted_iota(jnp.int32, (N, TB), 0)
    eq = s_iota == x[None, :]
    onehot_bf = jnp.where(eq, jnp.float32(1.0), jnp.float32(0.0)).astype(jnp.bfloat16)
    ut_bf = ut_ref[...].astype(jnp.bfloat16)
    a = jnp.broadcast_to(onehot_bf[:, None, :], (N, U, TB)).reshape(N * U, TB)
    b = jnp.broadcast_to(ut_bf[None, :, :], (N, U, TB)).reshape(N * U, TB)
    zt = jnp.concatenate([a * b, onehot_bf], axis=0)
    dn = (((0,), (0,)), ((), ()))
    logits = jax.lax.dot_general(zt, wm_ref[...], dn,
                                 preferred_element_type=jnp.float32)
    m = jnp.max(logits, axis=1, keepdims=True)
    ex = jnp.exp(logits - m)
    lz = jnp.log(jnp.sum(ex, axis=1, keepdims=True)) + m
    o_ref[...] = logits - lz


@functools.partial(jax.jit, static_argnames=("interpret", "tb"))
def kernel(x_curr, u_curr, logP0, W, interpret=False, tb=2048):
    T = x_curr.shape[0]
    N = logP0.shape[0]
    U = u_curr.shape[1]
    cols = jnp.arange(N)[None, :]
    srows = jnp.arange(N)[:, None]
    k = jnp.clip(cols - (cols > srows).astype(jnp.int32), 0, N - 2)
    W64 = jnp.take_along_axis(W, k[:, :, None], axis=1)
    W64 = jnp.where((cols == srows)[:, :, None], 0.0, W64)
    Wm = W64.transpose(0, 2, 1).reshape(N * U, N)
    Wtot = jnp.concatenate([Wm, logP0], axis=0).astype(jnp.bfloat16)

    TB = tb
    NB = T // TB
    x3 = x_curr.astype(jnp.int32).reshape(NB, 1, TB)
    uT = u_curr.T
    out = pl.pallas_call(
        functools.partial(_body, TB=TB, N=N, U=U),
        grid=(NB,),
        in_specs=[
            pl.BlockSpec((1, 1, TB), lambda i: (i, 0, 0)),
            pl.BlockSpec((U, TB), lambda i: (0, i)),
            pl.BlockSpec((N * U + N, N), lambda i: (0, 0)),
        ],
        out_specs=pl.BlockSpec((TB, N), lambda i: (i, 0)),
        out_shape=jax.ShapeDtypeStruct((T, N), jnp.float32),
        compiler_params=pltpu.CompilerParams(
            dimension_semantics=("arbitrary",),
        ),
        interpret=interpret,
    )(x3, uT, Wtot)
    return out
